# baseline (speedup 1.0000x reference)
import functools

import jax
import jax.numpy as jnp
from jax import lax
from jax.experimental import pallas as pl
from jax.experimental.pallas import tpu as pltpu
from jax.experimental.pallas import tpu_sc as plsc

NC = 2
NS = 16
NW = NC * NS
BN_EPS = 1e-5


@functools.lru_cache(maxsize=None)
def _seg_sum_kernel(N, E, D, colsplit=False):
    NPART = NS if colsplit else NW
    EC = E // NPART
    C = 96
    NCH = EC // C
    HN = NC * N if colsplit else N
    assert EC % C == 0 and NCH % 6 == 0
    RPT = (N // NS) // 8 * 8
    ZC = 104
    REM = N - NS * RPT
    assert EC % C == 0 and RPT % ZC == 0 and D % 16 == 0 and REM % 8 == 0
    assert REM <= ZC

    mesh = plsc.VectorSubcoreMesh(
        core_axis_name="c", subcore_axis_name="s",
        num_cores=NC, num_subcores=NS)

    def body(h_hbm, src_hbm, dst_hbm, w_hbm, out_hbm,
             sidx, didx, ew, gbuf0, gbuf1, gbuf2, sbuf0, sbuf1, zbuf, acc,
             gsem0, gsem1, gsem2, ssem0, ssem1):
        c = lax.axis_index("c")
        s = lax.axis_index("s")
        wid = s if colsplit else c * NS + s
        zero16 = jnp.zeros((16,), jnp.float32)
        gsems = (gsem0, gsem1, gsem2)
        ssems = (ssem0, ssem1)

        pltpu.async_copy(src_hbm.at[wid], sidx, gsem0)
        pltpu.async_copy(dst_hbm.at[wid], didx, gsem1)
        pltpu.async_copy(w_hbm.at[wid], ew, ssem0)

        def zrow(r, _):
            for j in range(D // 16):
                zbuf[r, pl.ds(j * 16, 16)] = zero16
            return 0
        lax.fori_loop(0, ZC, zrow, 0)
        row0 = s * RPT
        for k in range(RPT // ZC):
            pltpu.async_copy(zbuf, acc.at[pl.ds(row0 + k * ZC, ZC)], ssem1)

        @pl.when(s == NS - 1)
        def _():
            pltpu.async_copy(zbuf.at[pl.ds(0, REM)],
                             acc.at[pl.ds(NS * RPT, REM)], ssem1)
        for k in range(RPT // ZC):
            pltpu.make_async_copy(zbuf, acc.at[pl.ds(row0 + k * ZC, ZC)],
                                  ssem1).wait()

        @pl.when(s == NS - 1)
        def _():
            pltpu.make_async_copy(zbuf.at[pl.ds(0, REM)],
                                  acc.at[pl.ds(NS * RPT, REM)], ssem1).wait()
        pltpu.make_async_copy(src_hbm.at[wid], sidx, gsem0).wait()
        pltpu.make_async_copy(dst_hbm.at[wid], didx, gsem1).wait()
        pltpu.make_async_copy(w_hbm.at[wid], ew, ssem0).wait()
        plsc.subcore_barrier()
        if colsplit:
            def offs(g, _):
                for j in range(C // 16):
                    sidx[g, pl.ds(j * 16, 16)] = (
                        sidx[g, pl.ds(j * 16, 16)] * 2 + c)
                return 0
            lax.fori_loop(0, NCH, offs, 0)

        gbufs = (gbuf0, gbuf1, gbuf2)
        sbufs = (sbuf0, sbuf1)

        def start_gather(i, b):
            pltpu.async_copy(h_hbm.at[sidx.at[i]], gbufs[b], gsems[b])

        def wait_gather(i, b):
            pltpu.make_async_copy(h_hbm.at[sidx.at[i]], gbufs[b],
                                  gsems[b]).wait()

        def start_scatter(i, b):
            pltpu.async_copy(sbufs[b], acc.at[didx.at[i]], ssems[b], add=True)

        def wait_scatter(i, b):
            pltpu.make_async_copy(sbufs[b], acc.at[didx.at[i]],
                                  ssems[b]).wait()

        def scale(i, gbi, sbi):
            gb, sb = gbufs[gbi], sbufs[sbi]

            def mgrp(g, _):
                nv = ew[i, pl.ds(g * 16, 16)]
                for rr in range(16):
                    r = g * 16 + rr
                    sc = nv[rr]
                    for j in range(D // 16):
                        sb[r, pl.ds(j * 16, 16)] = (
                            gb[r, pl.ds(j * 16, 16)] * sc)
                return 0
            lax.fori_loop(0, C // 16, mgrp, 0)

        start_gather(0, 0)
        start_gather(1, 1)

        def pipe(p, _):
            for b6 in range(6):
                i = 6 * p + b6
                gb = b6 % 3
                sb = b6 % 2
                wait_gather(i, gb)

                @pl.when(i + 2 < NCH)
                def _():
                    start_gather(i + 2, (b6 + 2) % 3)

                @pl.when(i >= 2)
                def _():
                    wait_scatter(i - 2, sb)
                scale(i, gb, sb)
                start_scatter(i, sb)
            return 0
        lax.fori_loop(0, NCH // 6, pipe, 0)

        wait_scatter(NCH - 2, 0)
        wait_scatter(NCH - 1, 1)
        plsc.subcore_barrier()

        for k in range(RPT // ZC):
            r0 = row0 + k * ZC
            pltpu.async_copy(acc.at[pl.ds(r0, ZC)],
                             out_hbm.at[pl.ds(c * N + r0, ZC)], gsems[k % 2])

        @pl.when(s == NS - 1)
        def _():
            r0 = NS * RPT
            pltpu.async_copy(acc.at[pl.ds(r0, REM)],
                             out_hbm.at[pl.ds(c * N + r0, REM)], ssems[0])

        for k in range(RPT // ZC):
            r0 = row0 + k * ZC
            pltpu.make_async_copy(acc.at[pl.ds(r0, ZC)],
                                  out_hbm.at[pl.ds(c * N + r0, ZC)],
                                  gsems[k % 2]).wait()

        @pl.when(s == NS - 1)
        def _():
            r0 = NS * RPT
            pltpu.make_async_copy(acc.at[pl.ds(r0, REM)],
                                  out_hbm.at[pl.ds(c * N + r0, REM)],
                                  ssems[0]).wait()

    return pl.kernel(
        body,
        out_type=jax.ShapeDtypeStruct((NC * N, D), jnp.float32),
        mesh=mesh,
        scratch_types=[
            pltpu.VMEM((NCH, C), jnp.int32),
            pltpu.VMEM((NCH, C), jnp.int32),
            pltpu.VMEM((NCH, C), jnp.float32),
            pltpu.VMEM((C, D), jnp.float32),
            pltpu.VMEM((C, D), jnp.float32),
            pltpu.VMEM((C, D), jnp.float32),
            pltpu.VMEM((C, D), jnp.float32),
            pltpu.VMEM((C, D), jnp.float32),
            pltpu.VMEM((ZC, D), jnp.float32),
            pltpu.VMEM_SHARED((N, D), jnp.float32),
            pltpu.SemaphoreType.DMA,
            pltpu.SemaphoreType.DMA,
            pltpu.SemaphoreType.DMA,
            pltpu.SemaphoreType.DMA,
            pltpu.SemaphoreType.DMA,
        ],
        compiler_params=pltpu.CompilerParams(use_tc_tiling_on_sc=False),
        name=f"seg_sum_d{D}_{'col' if colsplit else 'edge'}",
    )


def _bn(t, g, be):
    mu = jnp.mean(t, axis=0, keepdims=True)
    var = jnp.mean((t - mu) ** 2, axis=0, keepdims=True)
    return g * (t - mu) * lax.rsqrt(var + BN_EPS) + be


def _stage_a(N):
    def body(p_ref, x_ref, w1_ref, dis_ref, hp_ref):
        deg = p_ref[0:N, 0:1] + p_ref[N:2 * N, 0:1] + 1.0
        dis = 1.0 / jnp.sqrt(deg)
        dis_ref[...] = dis
        hw = jnp.dot(x_ref[...], w1_ref[...],
                     preferred_element_type=jnp.float32)
        hp_ref[...] = hw * dis
    return body


def _stage_b(N, relu_first, split_out=False):
    def body(p_ref, hp_ref, dis_ref, b_ref, g_ref, be_ref, w_ref, *outs):
        dis = dis_ref[...]
        agg = p_ref[0:N, :] + p_ref[N:2 * N, :]
        y = dis * (agg + hp_ref[...]) + b_ref[...]
        if relu_first:
            h = _bn(jnp.maximum(y, 0.0), g_ref[...], be_ref[...])
        else:
            h = jnp.maximum(_bn(y, g_ref[...], be_ref[...]), 0.0)
        hw = jnp.dot(h, w_ref[...], preferred_element_type=jnp.float32)
        hp_next = hw * dis
        if split_out:
            D2 = hw.shape[1] // 2
            outs[0][...] = hp_next[:, :D2]
            outs[1][...] = hp_next[:, D2:]
        else:
            outs[0][...] = hp_next
    return body


def _prep_edges(N, E, EP, rows_in, rows_pad):
    def body(ei_ref, w_ref, srcp_ref, dstp_ref, wp_ref):
        pad = jax.lax.broadcasted_iota(jnp.int32, (rows_pad, 128), 0) * 128
        pad = pad + jax.lax.broadcasted_iota(jnp.int32, (rows_pad, 128), 1)
        pad = jax.lax.rem(pad, N)
        srcp_ref[0:rows_in, :] = ei_ref[0:rows_in, :]
        srcp_ref[rows_in:, :] = pad
        dstp_ref[0:rows_in, :] = ei_ref[rows_in:2 * rows_in, :]
        dstp_ref[rows_in:, :] = pad
        wp_ref[0:rows_in, :] = w_ref[...]
        wp_ref[rows_in:, :] = jnp.zeros((rows_pad, 128), jnp.float32)
    return body


def _stage_c(N):
    def body(pa_ref, pb_ref, ha_ref, hb_ref, dis_ref, b_ref, g_ref, be_ref,
             fc1w_ref, fc1b_ref, fc2w_ref, fc2b_ref, out_ref):
        dis = dis_ref[...]
        zacc = None
        for half, (p_ref, h_ref) in enumerate(((pa_ref, ha_ref),
                                               (pb_ref, hb_ref))):
            cols = pl.ds(64 * half, 64)
            agg = p_ref[0:N, :] + p_ref[N:2 * N, :]
            y = dis * (agg + h_ref[...]) + b_ref[0:1, cols]
            h = jnp.maximum(_bn(y, g_ref[0:1, cols], be_ref[0:1, cols]), 0.0)
            z = jnp.dot(h, fc1w_ref[cols, :],
                        preferred_element_type=jnp.float32)
            zacc = z if zacc is None else zacc + z
        z = jnp.maximum(zacc + fc1b_ref[...], 0.0)
        out_ref[...] = (jnp.dot(z, fc2w_ref[...],
                                preferred_element_type=jnp.float32)
                        + fc2b_ref[...])
    return body


def _tc_call(body, out_shapes, *args):
    return pl.pallas_call(
        body,
        out_shape=out_shapes,
    )(*args)


def kernel(x, edge_index, edge_attr,
           W1, b1, g1, be1, W2, b2, g2, be2, W3, b3, g3, be3,
           W4, b4, g4, be4, W5, b5, g5, be5,
           fc1_w, fc1_b, fc2_w, fc2_b):
    N = x.shape[0]
    E = edge_index.shape[1]
    C = 96
    EP = -(-E // (NW * C * 6)) * (NW * C * 6)
    rows_in = E // 128
    rows_pad = (EP - E) // 128
    srcp, dstp, wp = _tc_call(
        _prep_edges(N, E, EP, rows_in, rows_pad),
        (jax.ShapeDtypeStruct((EP // 128, 128), jnp.int32),
         jax.ShapeDtypeStruct((EP // 128, 128), jnp.int32),
         jax.ShapeDtypeStruct((EP // 128, 128), jnp.float32)),
        edge_index.reshape(2 * rows_in, 128),
        edge_attr.reshape(rows_in, 128))

    src3 = srcp.reshape(NW, -1, C)
    dst3 = dstp.reshape(NW, -1, C)
    w3 = wp.reshape(NW, -1, C)

    ones16 = jnp.ones((N, 16), jnp.float32)
    degp = _seg_sum_kernel(N, EP, 16)(ones16, src3, dst3, w3)

    dis, hp = _tc_call(
        _stage_a(N),
        (jax.ShapeDtypeStruct((N, 1), jnp.float32),
         jax.ShapeDtypeStruct((N, W1.shape[1]), jnp.float32)),
        degp, x, W1)


    layer_params = [
        (b1, g1, be1, W2, True),
        (b2, g2, be2, W3, True),
        (b3, g3, be3, W4, True),
        (b4, g4, be4, W5, False),
    ]
    for li, (b, g, be, Wn, relu_first) in enumerate(layer_params):
        p = _seg_sum_kernel(N, EP, hp.shape[1])(hp, src3, dst3, w3)
        last = li == len(layer_params) - 1
        out_shapes = (
            (jax.ShapeDtypeStruct((N, Wn.shape[1] // 2), jnp.float32),) * 2
            if last else
            jax.ShapeDtypeStruct((N, Wn.shape[1]), jnp.float32))
        hp = _tc_call(
            _stage_b(N, relu_first, split_out=last),
            out_shapes,
            p, hp, dis, b.reshape(1, -1), g.reshape(1, -1),
            be.reshape(1, -1), Wn)

    hp5a, hp5b = hp
    p5a = _seg_sum_kernel(N, EP, 64)(hp5a, src3, dst3, w3)
    p5b = _seg_sum_kernel(N, EP, 64)(hp5b, src3, dst3, w3)
    out = _tc_call(
        _stage_c(N),
        jax.ShapeDtypeStruct((N, 1), jnp.float32),
        p5a, p5b, hp5a, hp5b, dis,
        b5.reshape(1, -1), g5.reshape(1, -1), be5.reshape(1, -1),
        fc1_w, fc1_b.reshape(1, -1), fc2_w, fc2_b.reshape(1, -1))
    return out.reshape(-1)


# speedup vs baseline: 1.0097x; 1.0097x over previous
import functools

import jax
import jax.numpy as jnp
from jax import lax
from jax.experimental import pallas as pl
from jax.experimental.pallas import tpu as pltpu
from jax.experimental.pallas import tpu_sc as plsc

NC = 2
NS = 16
NW = NC * NS
BN_EPS = 1e-5


@functools.lru_cache(maxsize=None)
def _seg_sum_kernel(N, E, D, colsplit=False):
    NPART = NS if colsplit else NW
    EC = E // NPART
    C = 128
    NCH = EC // C
    HN = NC * N if colsplit else N
    assert EC % C == 0 and NCH % 2 == 0
    RPT = (N // NS) // 8 * 8
    ZC = 104
    REM = N - NS * RPT
    assert EC % C == 0 and RPT % ZC == 0 and D % 16 == 0 and REM % 8 == 0
    assert REM <= ZC

    mesh = plsc.VectorSubcoreMesh(
        core_axis_name="c", subcore_axis_name="s",
        num_cores=NC, num_subcores=NS)

    def body(h_hbm, src_hbm, dst_hbm, w_hbm, out_hbm,
             sidx, didx, ew, gbuf0, gbuf1, gbuf2, sbuf0, sbuf1, zbuf, acc,
             gsem0, gsem1, gsem2, ssem0, ssem1):
        c = lax.axis_index("c")
        s = lax.axis_index("s")
        wid = s if colsplit else c * NS + s
        zero16 = jnp.zeros((16,), jnp.float32)
        gsems = (gsem0, gsem1, gsem2)
        ssems = (ssem0, ssem1)

        pltpu.async_copy(src_hbm.at[wid], sidx, gsem0)
        pltpu.async_copy(dst_hbm.at[wid], didx, gsem1)
        pltpu.async_copy(w_hbm.at[wid], ew, ssem0)

        def zrow(r, _):
            for j in range(D // 16):
                zbuf[r, pl.ds(j * 16, 16)] = zero16
            return 0
        lax.fori_loop(0, ZC, zrow, 0)
        row0 = s * RPT
        for k in range(RPT // ZC):
            pltpu.async_copy(zbuf, acc.at[pl.ds(row0 + k * ZC, ZC)], ssem1)

        @pl.when(s == NS - 1)
        def _():
            pltpu.async_copy(zbuf.at[pl.ds(0, REM)],
                             acc.at[pl.ds(NS * RPT, REM)], ssem1)
        for k in range(RPT // ZC):
            pltpu.make_async_copy(zbuf, acc.at[pl.ds(row0 + k * ZC, ZC)],
                                  ssem1).wait()

        @pl.when(s == NS - 1)
        def _():
            pltpu.make_async_copy(zbuf.at[pl.ds(0, REM)],
                                  acc.at[pl.ds(NS * RPT, REM)], ssem1).wait()
        pltpu.make_async_copy(src_hbm.at[wid], sidx, gsem0).wait()
        pltpu.make_async_copy(dst_hbm.at[wid], didx, gsem1).wait()
        pltpu.make_async_copy(w_hbm.at[wid], ew, ssem0).wait()
        plsc.subcore_barrier()
        if colsplit:
            def offs(g, _):
                for j in range(C // 16):
                    sidx[g, pl.ds(j * 16, 16)] = (
                        sidx[g, pl.ds(j * 16, 16)] * 2 + c)
                return 0
            lax.fori_loop(0, NCH, offs, 0)

        gbufs = (gbuf0, gbuf1, gbuf2)
        sbufs = (sbuf0, sbuf1)

        def start_gather(i, b):
            pltpu.async_copy(h_hbm.at[sidx.at[i]], gbufs[b], gsems[b])

        def wait_gather(i, b):
            pltpu.make_async_copy(h_hbm.at[sidx.at[i]], gbufs[b],
                                  gsems[b]).wait()

        def start_scatter(i, b):
            pltpu.async_copy(sbufs[b], acc.at[didx.at[i]], ssems[b], add=True)

        def wait_scatter(i, b):
            pltpu.make_async_copy(sbufs[b], acc.at[didx.at[i]],
                                  ssems[b]).wait()

        def scale(i, gbi, sbi):
            gb, sb = gbufs[gbi], sbufs[sbi]

            def mgrp(g, _):
                nv = ew[i, pl.ds(g * 16, 16)]
                for rr in range(16):
                    r = g * 16 + rr
                    sc = nv[rr]
                    for j in range(D // 16):
                        sb[r, pl.ds(j * 16, 16)] = (
                            gb[r, pl.ds(j * 16, 16)] * sc)
                return 0
            lax.fori_loop(0, C // 16, mgrp, 0)

        start_gather(0, 0)
        start_gather(1, 1)

        def pipe(p, _):
            for b in range(2):
                i = 2 * p + b
                wait_gather(i, b)

                @pl.when(i >= 2)
                def _():
                    wait_scatter(i - 2, b)
                scale(i, b, b)
                start_scatter(i, b)

                @pl.when(i + 2 < NCH)
                def _():
                    start_gather(i + 2, b)
            return 0
        lax.fori_loop(0, NCH // 2, pipe, 0)

        wait_scatter(NCH - 2, 0)
        wait_scatter(NCH - 1, 1)
        plsc.subcore_barrier()

        for k in range(RPT // ZC):
            r0 = row0 + k * ZC
            pltpu.async_copy(acc.at[pl.ds(r0, ZC)],
                             out_hbm.at[pl.ds(c * N + r0, ZC)], gsems[k % 2])

        @pl.when(s == NS - 1)
        def _():
            r0 = NS * RPT
            pltpu.async_copy(acc.at[pl.ds(r0, REM)],
                             out_hbm.at[pl.ds(c * N + r0, REM)], ssems[0])

        for k in range(RPT // ZC):
            r0 = row0 + k * ZC
            pltpu.make_async_copy(acc.at[pl.ds(r0, ZC)],
                                  out_hbm.at[pl.ds(c * N + r0, ZC)],
                                  gsems[k % 2]).wait()

        @pl.when(s == NS - 1)
        def _():
            r0 = NS * RPT
            pltpu.make_async_copy(acc.at[pl.ds(r0, REM)],
                                  out_hbm.at[pl.ds(c * N + r0, REM)],
                                  ssems[0]).wait()

    return pl.kernel(
        body,
        out_type=jax.ShapeDtypeStruct((NC * N, D), jnp.float32),
        mesh=mesh,
        scratch_types=[
            pltpu.VMEM((NCH, C), jnp.int32),
            pltpu.VMEM((NCH, C), jnp.int32),
            pltpu.VMEM((NCH, C), jnp.float32),
            pltpu.VMEM((C, D), jnp.float32),
            pltpu.VMEM((C, D), jnp.float32),
            pltpu.VMEM((C, D), jnp.float32),
            pltpu.VMEM((C, D), jnp.float32),
            pltpu.VMEM((C, D), jnp.float32),
            pltpu.VMEM((ZC, D), jnp.float32),
            pltpu.VMEM_SHARED((N, D), jnp.float32),
            pltpu.SemaphoreType.DMA,
            pltpu.SemaphoreType.DMA,
            pltpu.SemaphoreType.DMA,
            pltpu.SemaphoreType.DMA,
            pltpu.SemaphoreType.DMA,
        ],
        compiler_params=pltpu.CompilerParams(use_tc_tiling_on_sc=False),
        name=f"seg_sum_d{D}_{'col' if colsplit else 'edge'}",
    )


def _bn(t, g, be):
    mu = jnp.mean(t, axis=0, keepdims=True)
    var = jnp.mean((t - mu) ** 2, axis=0, keepdims=True)
    return g * (t - mu) * lax.rsqrt(var + BN_EPS) + be


def _stage_a(N):
    def body(p_ref, x_ref, w1_ref, dis_ref, hp_ref):
        deg = p_ref[0:N, 0:1] + p_ref[N:2 * N, 0:1] + 1.0
        dis = 1.0 / jnp.sqrt(deg)
        dis_ref[...] = dis
        hw = jnp.dot(x_ref[...], w1_ref[...],
                     preferred_element_type=jnp.float32)
        hp_ref[...] = hw * dis
    return body


def _stage_b(N, relu_first, split_out=False):
    def body(p_ref, hp_ref, dis_ref, b_ref, g_ref, be_ref, w_ref, *outs):
        dis = dis_ref[...]
        agg = p_ref[0:N, :] + p_ref[N:2 * N, :]
        y = dis * (agg + hp_ref[...]) + b_ref[...]
        if relu_first:
            h = _bn(jnp.maximum(y, 0.0), g_ref[...], be_ref[...])
        else:
            h = jnp.maximum(_bn(y, g_ref[...], be_ref[...]), 0.0)
        hw = jnp.dot(h, w_ref[...], preferred_element_type=jnp.float32)
        hp_next = hw * dis
        if split_out:
            D2 = hw.shape[1] // 2
            outs[0][...] = hp_next[:, :D2]
            outs[1][...] = hp_next[:, D2:]
        else:
            outs[0][...] = hp_next
    return body


def _prep_edges(N, E, EP, rows_in, rows_pad):
    def body(ei_ref, w_ref, srcp_ref, dstp_ref, wp_ref):
        pad = jax.lax.broadcasted_iota(jnp.int32, (rows_pad, 128), 0) * 128
        pad = pad + jax.lax.broadcasted_iota(jnp.int32, (rows_pad, 128), 1)
        pad = jax.lax.rem(pad, N)
        srcp_ref[0:rows_in, :] = ei_ref[0:rows_in, :]
        srcp_ref[rows_in:, :] = pad
        dstp_ref[0:rows_in, :] = ei_ref[rows_in:2 * rows_in, :]
        dstp_ref[rows_in:, :] = pad
        wp_ref[0:rows_in, :] = w_ref[...]
        wp_ref[rows_in:, :] = jnp.zeros((rows_pad, 128), jnp.float32)
    return body


def _stage_c(N):
    def body(pa_ref, pb_ref, ha_ref, hb_ref, dis_ref, b_ref, g_ref, be_ref,
             fc1w_ref, fc1b_ref, fc2w_ref, fc2b_ref, out_ref):
        dis = dis_ref[...]
        zacc = None
        for half, (p_ref, h_ref) in enumerate(((pa_ref, ha_ref),
                                               (pb_ref, hb_ref))):
            cols = pl.ds(64 * half, 64)
            agg = p_ref[0:N, :] + p_ref[N:2 * N, :]
            y = dis * (agg + h_ref[...]) + b_ref[0:1, cols]
            h = jnp.maximum(_bn(y, g_ref[0:1, cols], be_ref[0:1, cols]), 0.0)
            z = jnp.dot(h, fc1w_ref[cols, :],
                        preferred_element_type=jnp.float32)
            zacc = z if zacc is None else zacc + z
        z = jnp.maximum(zacc + fc1b_ref[...], 0.0)
        out_ref[...] = (jnp.dot(z, fc2w_ref[...],
                                preferred_element_type=jnp.float32)
                        + fc2b_ref[...])
    return body


def _tc_call(body, out_shapes, *args):
    return pl.pallas_call(
        body,
        out_shape=out_shapes,
    )(*args)


def kernel(x, edge_index, edge_attr,
           W1, b1, g1, be1, W2, b2, g2, be2, W3, b3, g3, be3,
           W4, b4, g4, be4, W5, b5, g5, be5,
           fc1_w, fc1_b, fc2_w, fc2_b):
    N = x.shape[0]
    E = edge_index.shape[1]
    C = 128
    EP = -(-E // (NW * C * 2)) * (NW * C * 2)
    rows_in = E // 128
    rows_pad = (EP - E) // 128
    srcp, dstp, wp = _tc_call(
        _prep_edges(N, E, EP, rows_in, rows_pad),
        (jax.ShapeDtypeStruct((EP // 128, 128), jnp.int32),
         jax.ShapeDtypeStruct((EP // 128, 128), jnp.int32),
         jax.ShapeDtypeStruct((EP // 128, 128), jnp.float32)),
        edge_index.reshape(2 * rows_in, 128),
        edge_attr.reshape(rows_in, 128))

    src3 = srcp.reshape(NW, -1, C)
    dst3 = dstp.reshape(NW, -1, C)
    w3 = wp.reshape(NW, -1, C)

    ones16 = jnp.ones((N, 16), jnp.float32)
    degp = _seg_sum_kernel(N, EP, 16)(ones16, src3, dst3, w3)

    dis, hp = _tc_call(
        _stage_a(N),
        (jax.ShapeDtypeStruct((N, 1), jnp.float32),
         jax.ShapeDtypeStruct((N, W1.shape[1]), jnp.float32)),
        degp, x, W1)


    layer_params = [
        (b1, g1, be1, W2, True),
        (b2, g2, be2, W3, True),
        (b3, g3, be3, W4, True),
        (b4, g4, be4, W5, False),
    ]
    for li, (b, g, be, Wn, relu_first) in enumerate(layer_params):
        p = _seg_sum_kernel(N, EP, hp.shape[1])(hp, src3, dst3, w3)
        last = li == len(layer_params) - 1
        out_shapes = (
            (jax.ShapeDtypeStruct((N, Wn.shape[1] // 2), jnp.float32),) * 2
            if last else
            jax.ShapeDtypeStruct((N, Wn.shape[1]), jnp.float32))
        hp = _tc_call(
            _stage_b(N, relu_first, split_out=last),
            out_shapes,
            p, hp, dis, b.reshape(1, -1), g.reshape(1, -1),
            be.reshape(1, -1), Wn)

    hp5a, hp5b = hp
    p5a = _seg_sum_kernel(N, EP, 64)(hp5a, src3, dst3, w3)
    p5b = _seg_sum_kernel(N, EP, 64)(hp5b, src3, dst3, w3)
    out = _tc_call(
        _stage_c(N),
        jax.ShapeDtypeStruct((N, 1), jnp.float32),
        p5a, p5b, hp5a, hp5b, dis,
        b5.reshape(1, -1), g5.reshape(1, -1), be5.reshape(1, -1),
        fc1_w, fc1_b.reshape(1, -1), fc2_w, fc2_b.reshape(1, -1))
    return out.reshape(-1)
